# Initial kernel scaffold; baseline (speedup 1.0000x reference)
import jax
import jax.numpy as jnp
from jax.experimental import pallas as pl


def kernel(x, coords, params, nei_in, nei_out):
    raise NotImplementedError("write your pallas kernel here")


# speedup vs baseline: 8.2742x; 8.2742x over previous
import functools

import numpy as np
import jax
import jax.numpy as jnp
from jax import lax
from jax.experimental import pallas as pl
from jax.experimental.pallas import tpu as pltpu, tpu_sc as plsc

N = 10000
P = 512
V = 4
GROUPS = (27, 8, 8)
K = sum(GROUPS)
KPAD = 44
NPAD = 10240
COLS = tuple(i for g in GROUPS for i in range(g))
EPS = 1e-5
BR = 2000
NBR = N // BR
BE = 2048
NBE = NPAD // BE
FN = jnp.float32

_SC_MESH = plsc.VectorSubcoreMesh(core_axis_name="c", subcore_axis_name="s")
_SC_PARAMS = pltpu.CompilerParams(needs_layout_passes=False)


def _a_body(x_ref, c_ref, wq1, bq1, wv, bv, wp1, bp1, hq_o, vv_o, p1_o, st_o):
    xb = x_ref[...]
    hq = jnp.dot(xb, wq1[...], preferred_element_type=FN) + bq1[...]
    vv = jnp.dot(xb, wv[...], preferred_element_type=FN) + bv[...]
    p1 = jnp.dot(c_ref[...], wp1[...], preferred_element_type=FN) + bp1[...]
    hq_o[...] = hq
    vv_o[...] = vv
    p1_o[...] = p1
    st = jnp.concatenate([
        hq.sum(0, keepdims=True), (hq * hq).sum(0, keepdims=True),
        vv.sum(0, keepdims=True), (vv * vv).sum(0, keepdims=True),
        p1.sum(0, keepdims=True), (p1 * p1).sum(0, keepdims=True)], axis=0)

    @pl.when(pl.program_id(0) == 0)
    def _():
        st_o[...] = st

    @pl.when(pl.program_id(0) != 0)
    def _():
        st_o[...] = st_o[...] + st


def _bn_relu(y, s0, s1, g, be):
    m = s0 / float(N)
    v = s1 / float(N) - m * m
    return jnp.maximum(g * (y - m) / jnp.sqrt(v + EPS) + be, 0.0)


def _b_body(hq_ref, vv_ref, p1_ref, st_ref, gq1, beq1, gv, bev, gp1, bep1,
            wq2, bq2, wp2, bp2, vf_o, qp_o, pp_o, st2_o):
    st = st_ref[...]
    hq = _bn_relu(hq_ref[...], st[0:1], st[1:2], gq1[...], beq1[...])
    vf = _bn_relu(vv_ref[...], st[2:3], st[3:4], gv[...], bev[...])
    p1 = _bn_relu(p1_ref[...], st[4:5], st[5:6], gp1[...], bep1[...])
    qp = jnp.dot(hq, wq2[...], preferred_element_type=FN) + bq2[...]
    pp = jnp.dot(p1, wp2[...], preferred_element_type=FN) + bp2[...]
    vf_o[...] = vf
    qp_o[...] = qp
    pp_o[...] = pp
    st2 = jnp.concatenate([
        qp.sum(0, keepdims=True), (qp * qp).sum(0, keepdims=True),
        pp.sum(0, keepdims=True), (pp * pp).sum(0, keepdims=True)], axis=0)

    @pl.when(pl.program_id(0) == 0)
    def _():
        st2_o[...] = st2

    @pl.when(pl.program_id(0) != 0)
    def _():
        st2_o[...] = st2_o[...] + st2


def _c_body(vf_ref, qp_ref, pp_ref, st2_ref, gq2, beq2, gp2, bep2, exp_ref,
            qf_o, vpe_o):
    st2 = st2_ref[...]
    qf = _bn_relu(qp_ref[...], st2[0:1], st2[1:2], gq2[...], beq2[...])
    pos = _bn_relu(pp_ref[...], st2[2:3], st2[3:4], gp2[...], bep2[...])
    qf_o[...] = qf
    vpe_o[...] = vf_ref[...] + jnp.dot(pos, exp_ref[...],
                                       preferred_element_type=FN)


def _d_body(ni_h, no_h, qf_h, src_h, nq_h,
            ni_v, no_v, win_v, qf_v, nq_v):
    cid = lax.axis_index("c")
    sid = lax.axis_index("s")
    wid = sid * 2 + cid
    lane = lax.iota(jnp.int32, 16)
    pltpu.sync_copy(qf_h, qf_v)

    def do_k(k):
        pltpu.sync_copy(ni_h.at[pl.ds(pl.multiple_of(k * N, 8), N)], ni_v)
        pltpu.sync_copy(no_h.at[pl.ds(pl.multiple_of(k * N, 8), N)], no_v)

        def initb(i, c):
            win_v[pl.ds(i * 16, 16)] = jnp.full((16,), -1, jnp.int32)
            return c

        lax.fori_loop(0, N // 16, initb, 0)

        def pass1(i, c):
            d = no_v[pl.ds(i * 16, 16)]
            s = ni_v[pl.ds(i * 16, 16)]
            plsc.store_scatter(win_v, [d], s)
            return c

        lax.fori_loop(0, N // 16, pass1, 0)

        def pass2(i, c):
            w = win_v[pl.ds(i * 16, 16)]
            mf = (w >= 0).astype(FN)
            srcc = jnp.maximum(w, 0)
            nvec = i * 16 + lane
            for d in range(V):
                qg = plsc.load_gather(qf_v, [srcc * V + d])
                qn = plsc.load_gather(qf_v, [nvec * V + d])
                nq_v[pl.ds(d * NPAD + i * 16, 16)] = (qg - qn) * mf
            return c

        lax.fori_loop(0, N // 16, pass2, 0)
        pltpu.sync_copy(win_v,
                        src_h.at[pl.ds(pl.multiple_of(k * NPAD, 8), N)])
        pltpu.sync_copy(nq_v,
                        nq_h.at[pl.ds(pl.multiple_of(k * V * NPAD, 8),
                                      V * NPAD)])

    do_k(wid)

    @pl.when(wid < K - 32)
    def _():
        do_k(wid + 32)


def _e_body(nq_ref, src_ref, wm1, bm1, wm2, bm2, attn_o):
    nqb = nq_ref[...]
    msk = (src_ref[...] >= 0).astype(FN)
    ys = []
    for k in range(K):
        h = lax.dot_general(wm1[...], nqb[k], (((0,), (0,)), ((), ())),
                            preferred_element_type=FN) + bm1[...]
        h = jnp.maximum(h, 0.0)
        y = lax.dot_general(wm2[...], h, (((0,), (0,)), ((), ())),
                            preferred_element_type=FN) + bm2[...]
        ys.append(y * msk[k][None, :])
    yst = jnp.stack(ys, axis=0)
    mx = yst.max(axis=0)
    ex = jnp.exp(yst - mx)
    sm = ex.sum(axis=0)
    att = ex / sm
    for k in range(KPAD):
        if k < K:
            attn_o[k] = att[COLS[k]]
        else:
            attn_o[k] = jnp.zeros_like(att[0])


def _f_body(vpe_h, src_h, attn_h, out_h,
            src_v, attn_v, wt_v, idx_v, gbuf_v, acc_v, sem):
    cid = lax.axis_index("c")
    sid = lax.axis_index("s")
    wid = sid * 2 + cid
    lane = lax.iota(jnp.int32, 16)
    nchunks = NPAD // 128
    nmy = (nchunks - wid + 31) // 32

    def chunk_body(i, _):
        c = wid + i * 32
        r0 = pl.multiple_of(c * 128, 128)
        pltpu.sync_copy(src_h.at[:, pl.ds(r0, 128)], src_v)
        pltpu.sync_copy(attn_h.at[:, :, pl.ds(r0, 128)], attn_v)

        def sub_body(s, _s):
            s16 = s * 16

            def zb(z, cc):
                for r in range(16):
                    acc_v[r, pl.ds(z * 16, 16)] = jnp.zeros((16,), FN)
                return cc

            lax.fori_loop(0, P // 16, zb, 0)

            def pass_body(p, cc):
                for j in range(4):
                    k = p * 4 + j
                    sv = src_v[k, pl.ds(s16, 16)]
                    mf = (sv >= 0).astype(FN)
                    idxe = jnp.where(sv >= 0, sv, r0 + s16 + lane)
                    idxe = jnp.clip(idxe, 0, N - 1)
                    idx_v[pl.ds(j * 16, 16)] = idxe
                    for d in range(V):
                        wt_v[j, d] = attn_v[k, d, pl.ds(s16, 16)] * mf
                pltpu.async_copy(vpe_h.at[idx_v], gbuf_v, sem).wait()
                for d in range(V):
                    wvs = [wt_v[jj, d] for jj in range(4)]

                    def s2b(s2, c2, _wvs=wvs, _d=d):
                        c0 = _d * 128 + s2 * 16
                        for r in range(16):
                            a = acc_v[r, pl.ds(c0, 16)]
                            for jj in range(4):
                                a = a + _wvs[jj][r] * gbuf_v[jj * 16 + r,
                                                             pl.ds(c0, 16)]
                            acc_v[r, pl.ds(c0, 16)] = a
                        return c2

                    lax.fori_loop(0, 8, s2b, 0)
                return cc

            lax.fori_loop(0, KPAD // 4, pass_body, 0)
            pltpu.sync_copy(
                acc_v, out_h.at[pl.ds(pl.multiple_of(r0 + s16, 16), 16)])
            return _s

        lax.fori_loop(0, 8, sub_body, 0)
        return _

    lax.fori_loop(0, nmy, chunk_body, 0)


def _g_body(o_ref, st_o):
    ob = o_ref[...]
    st = jnp.concatenate([ob.sum(0, keepdims=True),
                          (ob * ob).sum(0, keepdims=True)], axis=0)

    @pl.when(pl.program_id(0) == 0)
    def _():
        st_o[...] = st

    @pl.when(pl.program_id(0) != 0)
    def _():
        st_o[...] = st_o[...] + st


def _h_body(o_ref, st_ref, go, beo, x_ref, out_o):
    st = st_ref[...]
    out_o[...] = _bn_relu(o_ref[...], st[0:1], st[1:2], go[...], beo[...]) \
        + x_ref[...]


def _row_spec(r, cdim):
    return pl.BlockSpec((BR, cdim), lambda i: (i, 0))


def _full_spec(shape):
    nd = len(shape)
    return pl.BlockSpec(shape, lambda i: (0,) * nd)


def kernel(x, coords, params, nei_in, nei_out):
    p = params
    r2 = lambda a: a.reshape(1, -1)

    hq_pre, v_pre, p1_pre, st = pl.pallas_call(
        _a_body,
        grid=(NBR,),
        in_specs=[
            _row_spec(BR, P), _row_spec(BR, 3),
            _full_spec((P, P)), _full_spec((1, P)),
            _full_spec((P, P)), _full_spec((1, P)),
            _full_spec((3, P)), _full_spec((1, P)),
        ],
        out_specs=[
            _row_spec(BR, P), _row_spec(BR, P), _row_spec(BR, P),
            _full_spec((6, P)),
        ],
        out_shape=[
            jax.ShapeDtypeStruct((N, P), FN),
            jax.ShapeDtypeStruct((N, P), FN),
            jax.ShapeDtypeStruct((N, P), FN),
            jax.ShapeDtypeStruct((6, P), FN),
        ],
    )(x, coords, p['W_q1'], r2(p['b_q1']), p['W_v'], r2(p['b_v']),
      p['W_p1'], r2(p['b_p1']))

    v_f, q_pre, p2_pre, st2 = pl.pallas_call(
        _b_body,
        grid=(NBR,),
        in_specs=[
            _row_spec(BR, P), _row_spec(BR, P), _row_spec(BR, P),
            _full_spec((6, P)),
            _full_spec((1, P)), _full_spec((1, P)),
            _full_spec((1, P)), _full_spec((1, P)),
            _full_spec((1, P)), _full_spec((1, P)),
            _full_spec((P, V)), _full_spec((1, V)),
            _full_spec((P, V)), _full_spec((1, V)),
        ],
        out_specs=[
            _row_spec(BR, P), _row_spec(BR, V), _row_spec(BR, V),
            _full_spec((4, V)),
        ],
        out_shape=[
            jax.ShapeDtypeStruct((N, P), FN),
            jax.ShapeDtypeStruct((N, V), FN),
            jax.ShapeDtypeStruct((N, V), FN),
            jax.ShapeDtypeStruct((4, V), FN),
        ],
    )(hq_pre, v_pre, p1_pre, st,
      r2(p['g_q1']), r2(p['be_q1']), r2(p['g_v']), r2(p['be_v']),
      r2(p['g_p1']), r2(p['be_p1']),
      p['W_q2'], r2(p['b_q2']), p['W_p2'], r2(p['b_p2']))

    expand = jnp.asarray(np.repeat(np.eye(V, dtype=np.float32),
                                   P // V, axis=1))
    q_f, v_pe = pl.pallas_call(
        _c_body,
        grid=(NBR,),
        in_specs=[
            _row_spec(BR, P), _row_spec(BR, V), _row_spec(BR, V),
            _full_spec((4, V)),
            _full_spec((1, V)), _full_spec((1, V)),
            _full_spec((1, V)), _full_spec((1, V)),
            _full_spec((V, P)),
        ],
        out_specs=[_row_spec(BR, V), _row_spec(BR, P)],
        out_shape=[
            jax.ShapeDtypeStruct((N, V), FN),
            jax.ShapeDtypeStruct((N, P), FN),
        ],
    )(v_f, q_pre, p2_pre, st2,
      r2(p['g_q2']), r2(p['be_q2']), r2(p['g_p2']), r2(p['be_p2']), expand)

    d_call = functools.partial(
        pl.kernel,
        out_type=(
            jax.ShapeDtypeStruct((KPAD * NPAD,), jnp.int32),
            jax.ShapeDtypeStruct((K * V * NPAD,), FN),
        ),
        mesh=_SC_MESH,
        compiler_params=_SC_PARAMS,
        scratch_types=[
            pltpu.VMEM((N,), jnp.int32),
            pltpu.VMEM((N,), jnp.int32),
            pltpu.VMEM((N,), jnp.int32),
            pltpu.VMEM((N * V,), FN),
            pltpu.VMEM((V * NPAD,), FN),
        ],
    )
    src1d, nq1d = d_call(_d_body)(nei_in.reshape(-1), nei_out.reshape(-1),
                                  q_f.reshape(-1))
    src_map = src1d.reshape(KPAD, NPAD)
    nq = nq1d.reshape(K, V, NPAD)

    attn = pl.pallas_call(
        _e_body,
        grid=(NBE,),
        in_specs=[
            pl.BlockSpec((K, V, BE), lambda i: (0, 0, i)),
            pl.BlockSpec((KPAD, BE), lambda i: (0, i)),
            _full_spec((V, P)), _full_spec((P, 1)),
            _full_spec((P, V)), _full_spec((V, 1)),
        ],
        out_specs=pl.BlockSpec((KPAD, V, BE), lambda i: (0, 0, i)),
        out_shape=jax.ShapeDtypeStruct((KPAD, V, NPAD), FN),
    )(nq, src_map, p['W_m1'], p['b_m1'].reshape(P, 1),
      p['W_m2'], p['b_m2'].reshape(V, 1))

    f_call = functools.partial(
        pl.kernel,
        out_type=jax.ShapeDtypeStruct((NPAD, P), FN),
        mesh=_SC_MESH,
        compiler_params=_SC_PARAMS,
        scratch_types=[
            pltpu.VMEM((KPAD, 128), jnp.int32),
            pltpu.VMEM((KPAD, V, 128), FN),
            pltpu.VMEM((4, V, 16), FN),
            pltpu.VMEM((64,), jnp.int32),
            pltpu.VMEM((64, P), FN),
            pltpu.VMEM((16, P), FN),
            pltpu.SemaphoreType.DMA,
        ],
    )
    out_pre = f_call(_f_body)(v_pe, src_map, attn)

    sto = pl.pallas_call(
        _g_body,
        grid=(NBR,),
        in_specs=[_row_spec(BR, P)],
        out_specs=_full_spec((2, P)),
        out_shape=jax.ShapeDtypeStruct((2, P), FN),
    )(out_pre)

    out = pl.pallas_call(
        _h_body,
        grid=(NBR,),
        in_specs=[
            _row_spec(BR, P), _full_spec((2, P)),
            _full_spec((1, P)), _full_spec((1, P)), _row_spec(BR, P),
        ],
        out_specs=_row_spec(BR, P),
        out_shape=jax.ShapeDtypeStruct((N, P), FN),
    )(out_pre, sto, r2(p['g_o']), r2(p['be_o']), x)

    return out


# speedup vs baseline: 10.1624x; 1.2282x over previous
import functools

import numpy as np
import jax
import jax.numpy as jnp
from jax import lax
from jax.experimental import pallas as pl
from jax.experimental.pallas import tpu as pltpu, tpu_sc as plsc

N = 10000
P = 512
V = 4
GROUPS = (27, 8, 8)
K = sum(GROUPS)
KPAD = 44
NPAD = 10240
COLS = tuple(i for g in GROUPS for i in range(g))
EPS = 1e-5
BR = 2000
NBR = N // BR
BE = 2048
NBE = NPAD // BE
FN = jnp.float32

_SC_MESH = plsc.VectorSubcoreMesh(core_axis_name="c", subcore_axis_name="s")
_SC_PARAMS = pltpu.CompilerParams(needs_layout_passes=False)


def _a_body(x_ref, c_ref, wq1, bq1, wv, bv, wp1, bp1, hq_o, vv_o, p1_o, st_o):
    xb = x_ref[...]
    hq = jnp.dot(xb, wq1[...], preferred_element_type=FN) + bq1[...]
    vv = jnp.dot(xb, wv[...], preferred_element_type=FN) + bv[...]
    p1 = jnp.dot(c_ref[...], wp1[...], preferred_element_type=FN) + bp1[...]
    hq_o[...] = hq
    vv_o[...] = vv
    p1_o[...] = p1
    st = jnp.concatenate([
        hq.sum(0, keepdims=True), (hq * hq).sum(0, keepdims=True),
        vv.sum(0, keepdims=True), (vv * vv).sum(0, keepdims=True),
        p1.sum(0, keepdims=True), (p1 * p1).sum(0, keepdims=True)], axis=0)

    @pl.when(pl.program_id(0) == 0)
    def _():
        st_o[...] = st

    @pl.when(pl.program_id(0) != 0)
    def _():
        st_o[...] = st_o[...] + st


def _bn_relu(y, s0, s1, g, be):
    m = s0 / float(N)
    v = s1 / float(N) - m * m
    return jnp.maximum(g * (y - m) / jnp.sqrt(v + EPS) + be, 0.0)


def _b_body(hq_ref, vv_ref, p1_ref, st_ref, gq1, beq1, gv, bev, gp1, bep1,
            wq2, bq2, wp2, bp2, vf_o, qp_o, pp_o, st2_o):
    st = st_ref[...]
    hq = _bn_relu(hq_ref[...], st[0:1], st[1:2], gq1[...], beq1[...])
    vf = _bn_relu(vv_ref[...], st[2:3], st[3:4], gv[...], bev[...])
    p1 = _bn_relu(p1_ref[...], st[4:5], st[5:6], gp1[...], bep1[...])
    qp = jnp.dot(hq, wq2[...], preferred_element_type=FN) + bq2[...]
    pp = jnp.dot(p1, wp2[...], preferred_element_type=FN) + bp2[...]
    vf_o[...] = vf
    qp_o[...] = qp
    pp_o[...] = pp
    st2 = jnp.concatenate([
        qp.sum(0, keepdims=True), (qp * qp).sum(0, keepdims=True),
        pp.sum(0, keepdims=True), (pp * pp).sum(0, keepdims=True)], axis=0)

    @pl.when(pl.program_id(0) == 0)
    def _():
        st2_o[...] = st2

    @pl.when(pl.program_id(0) != 0)
    def _():
        st2_o[...] = st2_o[...] + st2


def _c_body(vf_ref, qp_ref, pp_ref, st2_ref, gq2, beq2, gp2, bep2, exp_ref,
            qf_o, vpe_o):
    st2 = st2_ref[...]
    qf = _bn_relu(qp_ref[...], st2[0:1], st2[1:2], gq2[...], beq2[...])
    pos = _bn_relu(pp_ref[...], st2[2:3], st2[3:4], gp2[...], bep2[...])
    qf_o[...] = qf
    vpe_o[...] = vf_ref[...] + jnp.dot(pos, exp_ref[...],
                                       preferred_element_type=FN)


def _d_body(ni_h, no_h, qf_h, src_h, nq_h,
            ni_v, no_v, win_v, qf_v, nq_v):
    cid = lax.axis_index("c")
    sid = lax.axis_index("s")
    wid = sid * 2 + cid
    lane = lax.iota(jnp.int32, 16)
    pltpu.sync_copy(qf_h, qf_v)

    def do_k(k):
        pltpu.sync_copy(ni_h.at[pl.ds(pl.multiple_of(k * N, 8), N)], ni_v)
        pltpu.sync_copy(no_h.at[pl.ds(pl.multiple_of(k * N, 8), N)], no_v)

        def initb(i, c):
            win_v[pl.ds(i * 16, 16)] = jnp.full((16,), -1, jnp.int32)
            return c

        lax.fori_loop(0, N // 16, initb, 0)

        def pass1(i, c):
            d = no_v[pl.ds(i * 16, 16)]
            s = ni_v[pl.ds(i * 16, 16)]
            plsc.store_scatter(win_v, [d], s)
            return c

        lax.fori_loop(0, N // 16, pass1, 0)

        def pass2(i, c):
            w = win_v[pl.ds(i * 16, 16)]
            mf = (w >= 0).astype(FN)
            srcc = jnp.maximum(w, 0)
            nvec = i * 16 + lane
            for d in range(V):
                qg = plsc.load_gather(qf_v, [srcc * V + d])
                qn = plsc.load_gather(qf_v, [nvec * V + d])
                nq_v[pl.ds(d * NPAD + i * 16, 16)] = (qg - qn) * mf
            return c

        lax.fori_loop(0, N // 16, pass2, 0)
        pltpu.sync_copy(win_v,
                        src_h.at[pl.ds(pl.multiple_of(k * NPAD, 8), N)])
        pltpu.sync_copy(nq_v,
                        nq_h.at[pl.ds(pl.multiple_of(k * V * NPAD, 8),
                                      V * NPAD)])

    do_k(wid)

    @pl.when(wid < K - 32)
    def _():
        do_k(wid + 32)


def _e_body(nq_ref, src_ref, wm1, bm1, wm2, bm2, attn_o):
    nqb = nq_ref[...]
    msk = (src_ref[...] >= 0).astype(FN)
    ys = []
    for k in range(K):
        h = lax.dot_general(wm1[...], nqb[k], (((0,), (0,)), ((), ())),
                            preferred_element_type=FN) + bm1[...]
        h = jnp.maximum(h, 0.0)
        y = lax.dot_general(wm2[...], h, (((0,), (0,)), ((), ())),
                            preferred_element_type=FN) + bm2[...]
        ys.append(y * msk[k][None, :])
    yst = jnp.stack(ys, axis=0)
    mx = yst.max(axis=0)
    ex = jnp.exp(yst - mx)
    sm = ex.sum(axis=0)
    att = ex / sm
    for k in range(KPAD):
        if k < K:
            attn_o[k] = att[COLS[k]]
        else:
            attn_o[k] = jnp.zeros_like(att[0])


def _f_body(vpe_h, src_h, attn_h, out_h,
            src_v, attn_v, wt0_v, wt1_v, idx0_v, idx1_v,
            gbuf0_v, gbuf1_v, acc_v, sem0, sem1):
    cid = lax.axis_index("c")
    sid = lax.axis_index("s")
    wid = sid * 2 + cid
    lane = lax.iota(jnp.int32, 16)
    nchunks = NPAD // 128
    nmy = (nchunks - wid + 31) // 32
    NP = KPAD // 4
    bufs = ((idx0_v, wt0_v, gbuf0_v, sem0), (idx1_v, wt1_v, gbuf1_v, sem1))

    def chunk_body(i, _):
        c = wid + i * 32
        r0 = pl.multiple_of(c * 128, 128)
        pltpu.sync_copy(src_h.at[:, pl.ds(r0, 128)], src_v)
        pltpu.sync_copy(attn_h.at[:, :, pl.ds(r0, 128)], attn_v)

        def sub_body(s, _s):
            s16 = s * 16

            def zb(z, cc):
                for r in range(16):
                    acc_v[r, pl.ds(z * 16, 16)] = jnp.zeros((16,), FN)
                return cc

            lax.fori_loop(0, P // 16, zb, 0)

            def prep(p, idxb, wtb, gbufb, semb):
                for j in range(4):
                    k = p * 4 + j
                    sv = src_v[k, pl.ds(s16, 16)]
                    mf = (sv >= 0).astype(FN)
                    idxe = jnp.where(sv >= 0, sv, r0 + s16 + lane)
                    idxe = jnp.clip(idxe, 0, N - 1)
                    idxb[pl.ds(j * 16, 16)] = idxe
                    for d in range(V):
                        wtb[j, d] = attn_v[k, d, pl.ds(s16, 16)] * mf
                pltpu.async_copy(vpe_h.at[idxb], gbufb, semb)

            def fma(idxb, wtb, gbufb, semb):
                pltpu.make_async_copy(vpe_h.at[idxb], gbufb, semb).wait()
                for d in range(V):
                    wvs = [wtb[jj, d] for jj in range(4)]

                    def s2b(s2, c2, _wvs=wvs, _d=d):
                        c0 = _d * 128 + s2 * 16
                        for r in range(16):
                            a = acc_v[r, pl.ds(c0, 16)]
                            for jj in range(4):
                                a = a + _wvs[jj][r] * gbufb[jj * 16 + r,
                                                            pl.ds(c0, 16)]
                            acc_v[r, pl.ds(c0, 16)] = a
                        return c2

                    lax.fori_loop(0, 8, s2b, 0)

            prep(0, *bufs[0])

            def pass_body(p, cc):
                even = (p % 2) == 0
                nxt = p + 1 < NP

                @pl.when(jnp.logical_and(nxt, even))
                def _():
                    prep(p + 1, *bufs[1])

                @pl.when(jnp.logical_and(nxt, jnp.logical_not(even)))
                def _():
                    prep(p + 1, *bufs[0])

                @pl.when(even)
                def _():
                    fma(*bufs[0])

                @pl.when(jnp.logical_not(even))
                def _():
                    fma(*bufs[1])

                return cc

            lax.fori_loop(0, NP, pass_body, 0)
            pltpu.sync_copy(
                acc_v, out_h.at[pl.ds(pl.multiple_of(r0 + s16, 16), 16)])
            return _s

        lax.fori_loop(0, 8, sub_body, 0)
        return _

    lax.fori_loop(0, nmy, chunk_body, 0)


def _g_body(o_ref, st_o):
    ob = o_ref[...]
    st = jnp.concatenate([ob.sum(0, keepdims=True),
                          (ob * ob).sum(0, keepdims=True)], axis=0)

    @pl.when(pl.program_id(0) == 0)
    def _():
        st_o[...] = st

    @pl.when(pl.program_id(0) != 0)
    def _():
        st_o[...] = st_o[...] + st


def _h_body(o_ref, st_ref, go, beo, x_ref, out_o):
    st = st_ref[...]
    out_o[...] = _bn_relu(o_ref[...], st[0:1], st[1:2], go[...], beo[...]) \
        + x_ref[...]


def _row_spec(r, cdim):
    return pl.BlockSpec((BR, cdim), lambda i: (i, 0))


def _full_spec(shape):
    nd = len(shape)
    return pl.BlockSpec(shape, lambda i: (0,) * nd)


def kernel(x, coords, params, nei_in, nei_out):
    p = params
    r2 = lambda a: a.reshape(1, -1)

    hq_pre, v_pre, p1_pre, st = pl.pallas_call(
        _a_body,
        grid=(NBR,),
        in_specs=[
            _row_spec(BR, P), _row_spec(BR, 3),
            _full_spec((P, P)), _full_spec((1, P)),
            _full_spec((P, P)), _full_spec((1, P)),
            _full_spec((3, P)), _full_spec((1, P)),
        ],
        out_specs=[
            _row_spec(BR, P), _row_spec(BR, P), _row_spec(BR, P),
            _full_spec((6, P)),
        ],
        out_shape=[
            jax.ShapeDtypeStruct((N, P), FN),
            jax.ShapeDtypeStruct((N, P), FN),
            jax.ShapeDtypeStruct((N, P), FN),
            jax.ShapeDtypeStruct((6, P), FN),
        ],
    )(x, coords, p['W_q1'], r2(p['b_q1']), p['W_v'], r2(p['b_v']),
      p['W_p1'], r2(p['b_p1']))

    v_f, q_pre, p2_pre, st2 = pl.pallas_call(
        _b_body,
        grid=(NBR,),
        in_specs=[
            _row_spec(BR, P), _row_spec(BR, P), _row_spec(BR, P),
            _full_spec((6, P)),
            _full_spec((1, P)), _full_spec((1, P)),
            _full_spec((1, P)), _full_spec((1, P)),
            _full_spec((1, P)), _full_spec((1, P)),
            _full_spec((P, V)), _full_spec((1, V)),
            _full_spec((P, V)), _full_spec((1, V)),
        ],
        out_specs=[
            _row_spec(BR, P), _row_spec(BR, V), _row_spec(BR, V),
            _full_spec((4, V)),
        ],
        out_shape=[
            jax.ShapeDtypeStruct((N, P), FN),
            jax.ShapeDtypeStruct((N, V), FN),
            jax.ShapeDtypeStruct((N, V), FN),
            jax.ShapeDtypeStruct((4, V), FN),
        ],
    )(hq_pre, v_pre, p1_pre, st,
      r2(p['g_q1']), r2(p['be_q1']), r2(p['g_v']), r2(p['be_v']),
      r2(p['g_p1']), r2(p['be_p1']),
      p['W_q2'], r2(p['b_q2']), p['W_p2'], r2(p['b_p2']))

    expand = jnp.asarray(np.repeat(np.eye(V, dtype=np.float32),
                                   P // V, axis=1))
    q_f, v_pe = pl.pallas_call(
        _c_body,
        grid=(NBR,),
        in_specs=[
            _row_spec(BR, P), _row_spec(BR, V), _row_spec(BR, V),
            _full_spec((4, V)),
            _full_spec((1, V)), _full_spec((1, V)),
            _full_spec((1, V)), _full_spec((1, V)),
            _full_spec((V, P)),
        ],
        out_specs=[_row_spec(BR, V), _row_spec(BR, P)],
        out_shape=[
            jax.ShapeDtypeStruct((N, V), FN),
            jax.ShapeDtypeStruct((N, P), FN),
        ],
    )(v_f, q_pre, p2_pre, st2,
      r2(p['g_q2']), r2(p['be_q2']), r2(p['g_p2']), r2(p['be_p2']), expand)

    d_call = functools.partial(
        pl.kernel,
        out_type=(
            jax.ShapeDtypeStruct((KPAD * NPAD,), jnp.int32),
            jax.ShapeDtypeStruct((K * V * NPAD,), FN),
        ),
        mesh=_SC_MESH,
        compiler_params=_SC_PARAMS,
        scratch_types=[
            pltpu.VMEM((N,), jnp.int32),
            pltpu.VMEM((N,), jnp.int32),
            pltpu.VMEM((N,), jnp.int32),
            pltpu.VMEM((N * V,), FN),
            pltpu.VMEM((V * NPAD,), FN),
        ],
    )
    src1d, nq1d = d_call(_d_body)(nei_in.reshape(-1), nei_out.reshape(-1),
                                  q_f.reshape(-1))
    src_map = src1d.reshape(KPAD, NPAD)
    nq = nq1d.reshape(K, V, NPAD)

    attn = pl.pallas_call(
        _e_body,
        grid=(NBE,),
        in_specs=[
            pl.BlockSpec((K, V, BE), lambda i: (0, 0, i)),
            pl.BlockSpec((KPAD, BE), lambda i: (0, i)),
            _full_spec((V, P)), _full_spec((P, 1)),
            _full_spec((P, V)), _full_spec((V, 1)),
        ],
        out_specs=pl.BlockSpec((KPAD, V, BE), lambda i: (0, 0, i)),
        out_shape=jax.ShapeDtypeStruct((KPAD, V, NPAD), FN),
    )(nq, src_map, p['W_m1'], p['b_m1'].reshape(P, 1),
      p['W_m2'], p['b_m2'].reshape(V, 1))

    f_call = functools.partial(
        pl.kernel,
        out_type=jax.ShapeDtypeStruct((NPAD, P), FN),
        mesh=_SC_MESH,
        compiler_params=_SC_PARAMS,
        scratch_types=[
            pltpu.VMEM((KPAD, 128), jnp.int32),
            pltpu.VMEM((KPAD, V, 128), FN),
            pltpu.VMEM((4, V, 16), FN),
            pltpu.VMEM((4, V, 16), FN),
            pltpu.VMEM((64,), jnp.int32),
            pltpu.VMEM((64,), jnp.int32),
            pltpu.VMEM((64, P), FN),
            pltpu.VMEM((64, P), FN),
            pltpu.VMEM((16, P), FN),
            pltpu.SemaphoreType.DMA,
            pltpu.SemaphoreType.DMA,
        ],
    )
    out_pre = f_call(_f_body)(v_pe, src_map, attn)

    sto = pl.pallas_call(
        _g_body,
        grid=(NBR,),
        in_specs=[_row_spec(BR, P)],
        out_specs=_full_spec((2, P)),
        out_shape=jax.ShapeDtypeStruct((2, P), FN),
    )(out_pre)

    out = pl.pallas_call(
        _h_body,
        grid=(NBR,),
        in_specs=[
            _row_spec(BR, P), _full_spec((2, P)),
            _full_spec((1, P)), _full_spec((1, P)), _row_spec(BR, P),
        ],
        out_specs=_row_spec(BR, P),
        out_shape=jax.ShapeDtypeStruct((N, P), FN),
    )(out_pre, sto, r2(p['g_o']), r2(p['be_o']), x)

    return out
